# baseline (speedup 1.0000x reference)
import dataclasses
import functools

import jax
import jax.numpy as jnp
from jax import lax
from jax.experimental import pallas as pl
from jax.experimental.pallas import tpu as pltpu
from jax.experimental.pallas import tpu_sc as plsc

_VPAD = 128
_NC, _NS = 2, 16
_NW = _NC * _NS
_LANES = 16
_CHUNK = 128
_CSL = 56
_MOFF = 56
_OSL = 112


@functools.partial(jax.jit, static_argnames=("ncol",))
def _sc_call(idx, scores, tabs, *, ncol):
    nrow = idx.shape[0]
    rows_w = nrow // _NW
    nch = rows_w // _CHUNK
    assert nch % 2 == 0
    mesh = plsc.VectorSubcoreMesh(core_axis_name="c", subcore_axis_name="s")
    cp = pltpu.CompilerParams()
    if "needs_layout_passes" in pltpu.CompilerParams.__dataclass_fields__:
        cp = dataclasses.replace(cp, needs_layout_passes=False)
    cp = dataclasses.replace(cp, use_tc_tiling_on_sc=False)

    @functools.partial(
        pl.kernel,
        out_type=jax.ShapeDtypeStruct((nrow, 128), jnp.float32),
        mesh=mesh,
        scratch_types=[
            pltpu.VMEM((2, _CHUNK, _CSL), jnp.int32),
            pltpu.VMEM((2, _CHUNK, _CSL), jnp.float32),
            pltpu.VMEM((2, _CHUNK, _OSL), jnp.float32),
            pltpu.VMEM((5, _VPAD), jnp.float32),
            pltpu.SemaphoreType.DMA,
            pltpu.SemaphoreType.DMA,
            pltpu.SemaphoreType.DMA,
        ],
        compiler_params=cp,
    )
    def body(idx_hbm, s_hbm, tab_hbm, out_hbm,
             idx_v, s_v, o_v, tab_v, sem_a, sem_b, sem_o):
        wid = lax.axis_index("s") * _NC + lax.axis_index("c")
        row0 = wid * rows_w
        csl = pl.ds(0, _CSL)
        sems = (sem_a, sem_b)

        def rsl(ch):
            return pl.ds(row0 + ch * _CHUNK, _CHUNK)

        def start_in(ch, buf):
            a = pltpu.async_copy(idx_hbm.at[rsl(ch), csl], idx_v.at[buf],
                                 sems[buf])
            b = pltpu.async_copy(s_hbm.at[rsl(ch), csl], s_v.at[buf],
                                 sems[buf])
            return a, b

        in0 = start_in(0, 0)
        pltpu.sync_copy(tab_hbm, tab_v)
        in1 = start_in(1, 1)

        @pl.loop(0, _VPAD, step=_LANES)
        def _(i):
            sl = pl.ds(i, _LANES)
            tab_v[1, sl] = -(tab_v[1, sl] * tab_v[2, sl] + tab_v[3, sl])
            tab_v[0, sl] = -tab_v[0, sl]

        col_starts = list(range(0, ncol - _LANES, _LANES)) + [ncol - _LANES]

        na_t = tab_v.at[0]
        nc_t = tab_v.at[1]
        th_t = tab_v.at[4]

        def compute(buf):
            @plsc.parallel_loop(0, _CHUNK, unroll=4)
            def _(r):
                for c in col_starts:
                    sl = pl.ds(c, _LANES)
                    msl = pl.ds(_MOFF + c, _LANES)
                    ids = idx_v[buf, r, sl]
                    na = plsc.load_gather(na_t, [ids])
                    nc2 = plsc.load_gather(nc_t, [ids])
                    th = plsc.load_gather(th_t, [ids])
                    e = jnp.exp(na * s_v[buf, r, sl] + nc2)
                    cal = 1.0 / (1.0 + e)
                    o_v[buf, r, sl] = cal
                    o_v[buf, r, msl] = jnp.where(cal > th, jnp.float32(1.0),
                                                 jnp.float32(0.0))

        def drain_out(descs):
            for d in descs:
                d.wait()

        osl = pl.ds(0, _OSL)
        prev_out = ()
        for ch in range(nch):
            buf = ch % 2
            cur_in = in0 if buf == 0 else in1
            for d in cur_in:
                d.wait()
            compute(buf)
            drain_out(prev_out)
            oc = pltpu.async_copy(o_v.at[buf], out_hbm.at[rsl(ch), osl],
                                  sem_o)
            prev_out = (oc,)
            if ch + 2 < nch:
                nxt = start_in(ch + 2, buf)
                if buf == 0:
                    in0 = nxt
                else:
                    in1 = nxt
        drain_out(prev_out)

    return body(idx, scores, tabs)


def kernel(type_ids, scores, prior, weights, bias, threshold):
    v = prior.shape[0]
    pad = _VPAD - v
    ncol = type_ids.shape[1]
    b = type_ids.shape[0]
    zi = jnp.zeros((b, 128 - ncol), jnp.int32)
    idx = jnp.concatenate([type_ids.astype(jnp.int32), zi], axis=1)
    s = jnp.concatenate([scores, zi.astype(jnp.float32)], axis=1)
    tabs = jnp.pad(
        jnp.stack([weights[:, 0], weights[:, 1], prior, bias, threshold]),
        ((0, 0), (0, pad)))
    out = _sc_call(idx, s, tabs, ncol=ncol)
    return out[:, :ncol], (out[:, _MOFF:_MOFF + ncol] != 0.0)


# speedup vs baseline: 1.0110x; 1.0110x over previous
import dataclasses
import functools

import jax
import jax.numpy as jnp
from jax import lax
from jax.experimental import pallas as pl
from jax.experimental.pallas import tpu as pltpu
from jax.experimental.pallas import tpu_sc as plsc

_VPAD = 128
_NC, _NS = 2, 16
_NW = _NC * _NS
_LANES = 16
_CHUNK = 128
_CSL = 56


@functools.partial(jax.jit, static_argnames=("ncol",))
def _sc_call(idx, scores, tabs, *, ncol):
    nrow = idx.shape[0]
    rows_w = nrow // _NW
    nch = rows_w // _CHUNK
    assert nch % 2 == 0
    mesh = plsc.VectorSubcoreMesh(core_axis_name="c", subcore_axis_name="s")
    cp = pltpu.CompilerParams()
    if "needs_layout_passes" in pltpu.CompilerParams.__dataclass_fields__:
        cp = dataclasses.replace(cp, needs_layout_passes=False)
    cp = dataclasses.replace(cp, use_tc_tiling_on_sc=False)

    @functools.partial(
        pl.kernel,
        out_type=[
            jax.ShapeDtypeStruct((nrow, 128), jnp.float32),
            jax.ShapeDtypeStruct((nrow, 128), jnp.int32),
        ],
        mesh=mesh,
        scratch_types=[
            pltpu.VMEM((2, _CHUNK, _CSL), jnp.int32),
            pltpu.VMEM((2, _CHUNK, _CSL), jnp.float32),
            pltpu.VMEM((2, _CHUNK, _CSL), jnp.float32),
            pltpu.VMEM((2, _CHUNK, _CSL), jnp.int32),
            pltpu.VMEM((5, _VPAD), jnp.float32),
            pltpu.SemaphoreType.DMA,
            pltpu.SemaphoreType.DMA,
            pltpu.SemaphoreType.DMA,
        ],
        compiler_params=cp,
    )
    def body(idx_hbm, s_hbm, tab_hbm, cal_hbm, mask_hbm,
             idx_v, s_v, cal_v, m_v, tab_v, sem_a, sem_b, sem_o):
        wid = lax.axis_index("s") * _NC + lax.axis_index("c")
        row0 = wid * rows_w
        csl = pl.ds(0, _CSL)
        sems = (sem_a, sem_b)

        def rsl(ch):
            return pl.ds(row0 + ch * _CHUNK, _CHUNK)

        def start_in(ch, buf):
            a = pltpu.async_copy(idx_hbm.at[rsl(ch), csl], idx_v.at[buf],
                                 sems[buf])
            b = pltpu.async_copy(s_hbm.at[rsl(ch), csl], s_v.at[buf],
                                 sems[buf])
            return a, b

        in0 = start_in(0, 0)
        pltpu.sync_copy(tab_hbm, tab_v)
        in1 = start_in(1, 1)

        @pl.loop(0, _VPAD, step=_LANES)
        def _(i):
            sl = pl.ds(i, _LANES)
            tab_v[1, sl] = -(tab_v[1, sl] * tab_v[2, sl] + tab_v[3, sl])
            tab_v[0, sl] = -tab_v[0, sl]

        col_starts = list(range(0, ncol - _LANES, _LANES)) + [ncol - _LANES]

        na_t = tab_v.at[0]
        nc_t = tab_v.at[1]
        th_t = tab_v.at[4]

        def compute(buf):
            @plsc.parallel_loop(0, _CHUNK, unroll=4)
            def _(r):
                for c in col_starts:
                    sl = pl.ds(c, _LANES)
                    ids = idx_v[buf, r, sl]
                    na = plsc.load_gather(na_t, [ids])
                    nc2 = plsc.load_gather(nc_t, [ids])
                    th = plsc.load_gather(th_t, [ids])
                    e = jnp.exp(na * s_v[buf, r, sl] + nc2)
                    cal = 1.0 / (1.0 + e)
                    cal_v[buf, r, sl] = cal
                    m_v[buf, r, sl] = jnp.where(cal > th, jnp.int32(1),
                                                jnp.int32(0))

        def drain_out(descs):
            for d in descs:
                d.wait()

        prev_out = ()
        for ch in range(nch):
            buf = ch % 2
            cur_in = in0 if buf == 0 else in1
            for d in cur_in:
                d.wait()
            compute(buf)
            drain_out(prev_out)
            oc = pltpu.async_copy(cal_v.at[buf], cal_hbm.at[rsl(ch), csl],
                                  sem_o)
            om = pltpu.async_copy(m_v.at[buf], mask_hbm.at[rsl(ch), csl],
                                  sem_o)
            prev_out = (oc, om)
            if ch + 2 < nch:
                nxt = start_in(ch + 2, buf)
                if buf == 0:
                    in0 = nxt
                else:
                    in1 = nxt
        drain_out(prev_out)

    return body(idx, scores, tabs)


def kernel(type_ids, scores, prior, weights, bias, threshold):
    v = prior.shape[0]
    pad = _VPAD - v
    ncol = type_ids.shape[1]
    b = type_ids.shape[0]
    zi = jnp.zeros((b, 128 - ncol), jnp.int32)
    idx = jnp.concatenate([type_ids.astype(jnp.int32), zi], axis=1)
    s = jnp.concatenate([scores, zi.astype(jnp.float32)], axis=1)
    tabs = jnp.pad(
        jnp.stack([weights[:, 0], weights[:, 1], prior, bias, threshold]),
        ((0, 0), (0, pad)))
    cal, mask = _sc_call(idx, s, tabs, ncol=ncol)
    return cal[:, :ncol], mask[:, :ncol].astype(jnp.bool_)
